# baseline (speedup 1.0000x reference)
import functools
import math

import jax
import jax.numpy as jnp
from jax.experimental import pallas as pl
from jax.experimental.pallas import tpu as pltpu

DIM_ = 128
HALF_ = 64
BASE_ = 10000.0


def _round_to_f16(v):
    i = jax.lax.bitcast_convert_type(v, jnp.int32)
    bias = 0x0FFF + ((i >> 13) & 1)
    r = (i + bias) & jnp.int32(~0x1FFF)
    return jax.lax.bitcast_convert_type(r, jnp.float32)


def _rope_body(nS, pos_ref, cos_ref, sin_ref, cos_s, sin_s):
    h = pl.program_id(1)

    @pl.when(h == 0)
    def _compute():
        pos = pos_ref[0, :, :].astype(jnp.float32)
        i = jax.lax.broadcasted_iota(jnp.int32, (1, HALF_), 1).astype(jnp.float32)
        inv_freq = jnp.exp(i * (-math.log(BASE_) / HALF_))
        freqs = pos * inv_freq
        emb = jnp.concatenate([freqs, freqs], axis=-1)
        cos_s[...] = _round_to_f16(jnp.cos(emb))
        sin_s[...] = _round_to_f16(jnp.sin(emb))

    cos_ref[0, 0, :, :] = cos_s[...]
    sin_ref[0, 0, :, :] = sin_s[...]


@functools.partial(jax.jit, static_argnames=("interpret",))
def kernel(x, position_ids, interpret=False):
    B, H, S, D = x.shape
    bs = 512
    nS = S // bs
    pos3 = position_ids.reshape(B * nS, bs, 1)

    out_shape = jax.ShapeDtypeStruct((B, H, S, D), x.dtype)
    grid = (B * nS, H)

    def pos_map(i, h):
        return (i, 0, 0)

    def out_map(i, h):
        return (i // nS, h, i % nS, 0)

    cos, sin = pl.pallas_call(
        functools.partial(_rope_body, nS),
        grid=grid,
        in_specs=[pl.BlockSpec((1, bs, 1), pos_map)],
        out_specs=[
            pl.BlockSpec((1, 1, bs, D), out_map),
            pl.BlockSpec((1, 1, bs, D), out_map),
        ],
        out_shape=[out_shape, out_shape],
        scratch_shapes=[
            pltpu.VMEM((bs, D), jnp.float32),
            pltpu.VMEM((bs, D), jnp.float32),
        ],
        interpret=interpret,
    )(pos3)
    return cos, sin


# speedup vs baseline: 1.8732x; 1.8732x over previous
import functools
import math

import jax
import jax.numpy as jnp
from jax.experimental import pallas as pl
from jax.experimental.pallas import tpu as pltpu

DIM_ = 128
HALF_ = 64
BASE_ = 10000.0


def _round_to_f16(v):
    i = jax.lax.bitcast_convert_type(v, jnp.int32)
    bias = 0x0FFF + ((i >> 13) & 1)
    r = (i + bias) & jnp.int32(~0x1FFF)
    return jax.lax.bitcast_convert_type(r, jnp.float32)


def _compact_body(pos_ref, cos_ref, sin_ref):
    pos = pos_ref[0, :, :].astype(jnp.float32)
    i = jax.lax.broadcasted_iota(jnp.int32, (1, HALF_), 1).astype(jnp.float32)
    inv_freq = jnp.exp(i * (-math.log(BASE_) / HALF_))
    freqs = pos * inv_freq
    c = _round_to_f16(jnp.cos(freqs))
    s = _round_to_f16(jnp.sin(freqs))
    cos_ref[0, :, :] = jnp.concatenate([c, c], axis=-1)
    sin_ref[0, :, :] = jnp.concatenate([s, s], axis=-1)


def _bcast_body(hg, cos_c_ref, sin_c_ref, cos_ref, sin_ref):
    shape = cos_ref.shape
    cos_ref[...] = jnp.broadcast_to(cos_c_ref[...][None, :, :, :], shape)
    sin_ref[...] = jnp.broadcast_to(sin_c_ref[...][None, :, :, :], shape)


@functools.partial(jax.jit, static_argnames=("interpret",))
def kernel(x, position_ids, interpret=False):
    B, H, S, D = x.shape

    bs2 = 2048
    nS2 = S // bs2
    pos3 = position_ids.reshape(B * nS2, bs2, 1)
    compact_shape = jax.ShapeDtypeStruct((B, S, D), jnp.float32)
    cos_c, sin_c = pl.pallas_call(
        _compact_body,
        grid=(B * nS2,),
        in_specs=[pl.BlockSpec((1, bs2, 1), lambda i: (i, 0, 0))],
        out_specs=[
            pl.BlockSpec((1, bs2, D), lambda i: (i // nS2, i % nS2, 0)),
            pl.BlockSpec((1, bs2, D), lambda i: (i // nS2, i % nS2, 0)),
        ],
        out_shape=[compact_shape, compact_shape],
        interpret=interpret,
    )(pos3)

    bs = 1024
    hg = 2
    nS = S // bs
    out_shape = jax.ShapeDtypeStruct((B, H, S, D), x.dtype)

    def in_map(i, h):
        return (i // nS, i % nS, 0)

    def out_map(i, h):
        return (i // nS, h, i % nS, 0)

    cos, sin = pl.pallas_call(
        functools.partial(_bcast_body, hg),
        grid=(B * nS, H // hg),
        in_specs=[
            pl.BlockSpec((1, bs, D), in_map),
            pl.BlockSpec((1, bs, D), in_map),
        ],
        out_specs=[
            pl.BlockSpec((1, hg, bs, D), out_map),
            pl.BlockSpec((1, hg, bs, D), out_map),
        ],
        out_shape=[out_shape, out_shape],
        interpret=interpret,
    )(cos_c, sin_c)
    return cos, sin


# speedup vs baseline: 2.2483x; 1.2003x over previous
import functools
import math

import jax
import jax.numpy as jnp
from jax import lax
from jax.experimental import pallas as pl
from jax.experimental.pallas import tpu as pltpu
from jax.experimental.pallas import tpu_sc as plsc

DIM_ = 128
HALF_ = 64
BASE_ = 10000.0
CACHE_ = 4096

NC_ = 2
NS_ = 16
NW_ = NC_ * NS_


def _round_to_f16(v):
    i = jax.lax.bitcast_convert_type(v, jnp.int32)
    bias = 0x0FFF + ((i >> 13) & 1)
    r = (i + bias) & jnp.int32(~0x1FFF)
    return jax.lax.bitcast_convert_type(r, jnp.float32)


def _table_body(cos_ref, sin_ref):
    p = lax.broadcasted_iota(jnp.int32, (CACHE_, HALF_), 0).astype(jnp.float32)
    i = lax.broadcasted_iota(jnp.int32, (CACHE_, HALF_), 1).astype(jnp.float32)
    inv_freq = jnp.exp(i * (-math.log(BASE_) / HALF_))
    freqs = p * inv_freq
    c = _round_to_f16(jnp.cos(freqs))
    s = _round_to_f16(jnp.sin(freqs))
    cos_ref[...] = jnp.concatenate([c, c], axis=-1)
    sin_ref[...] = jnp.concatenate([s, s], axis=-1)


def _make_tables(interpret):
    t = jax.ShapeDtypeStruct((CACHE_, DIM_), jnp.float32)
    return pl.pallas_call(_table_body, out_shape=[t, t], interpret=interpret)()


def _sc_lookup(cos_t, sin_t, pos_flat, B, H, S, D):
    chunk = (B * S) // NW_
    mesh = plsc.VectorSubcoreMesh(
        core_axis_name="c", subcore_axis_name="s",
        num_cores=NC_, num_subcores=NS_)
    out_t = jax.ShapeDtypeStruct((B, H, S, D), jnp.float32)

    @functools.partial(
        pl.kernel,
        out_type=[out_t, out_t],
        mesh=mesh,
        scratch_types=[
            pltpu.VMEM((chunk,), jnp.int32),
            pltpu.VMEM((chunk, D), jnp.float32),
            pltpu.VMEM((chunk, D), jnp.float32),
            pltpu.SemaphoreType.DMA,
            pltpu.SemaphoreType.DMA,
        ],
    )
    def sc_kernel(cos_t_hbm, sin_t_hbm, pos_hbm, cos_out, sin_out,
                  idx_v, cos_v, sin_v, gsem, wsem):
        wid = lax.axis_index("s") * NC_ + lax.axis_index("c")
        base = wid * chunk
        b = base // S
        s0 = base % S
        pltpu.sync_copy(pos_hbm.at[pl.ds(base, chunk)], idx_v)
        g1 = pltpu.async_copy(cos_t_hbm.at[idx_v], cos_v, gsem)
        g2 = pltpu.async_copy(sin_t_hbm.at[idx_v], sin_v, gsem)
        g1.wait()
        g2.wait()
        grp = 4
        for g in range(0, H, grp):
            cps = []
            for h in range(g, g + grp):
                cps.append(pltpu.async_copy(
                    cos_v, cos_out.at[b, h, pl.ds(s0, chunk)], wsem))
                cps.append(pltpu.async_copy(
                    sin_v, sin_out.at[b, h, pl.ds(s0, chunk)], wsem))
            for cp in cps:
                cp.wait()

    return sc_kernel(cos_t, sin_t, pos_flat)


@functools.partial(jax.jit, static_argnames=("interpret",))
def kernel(x, position_ids, interpret=False):
    B, H, S, D = x.shape
    cos_t, sin_t = _make_tables(interpret)
    pos_flat = position_ids.reshape(B * S)
    cos, sin = _sc_lookup(cos_t, sin_t, pos_flat, B, H, S, D)
    return cos, sin
